# baseline (speedup 1.0000x reference)
import functools

import jax
import jax.numpy as jnp
import numpy as np
from jax import lax
from jax.experimental import pallas as pl
from jax.experimental.pallas import tpu as pltpu
from jax.experimental.pallas import tpu_sc as plsc

_B = 32
_T = 160000
_NR = 2 * _B
_NW = 32
_RPW = _NR // _NW
_CS = 32000
_CPR = _T // _CS
_NCH = _RPW * _CPR
_D = 4
_LAG = 2


def _np_threefry2x32(keypair, count):
    x0 = np.uint32(count[0]).copy()
    x1 = np.uint32(count[1]).copy()
    ks0, ks1 = np.uint32(keypair[0]), np.uint32(keypair[1])
    ks2 = np.uint32(0x1BD11BDA) ^ ks0 ^ ks1

    def rotl(x, d):
        return (x << np.uint32(d)) | (x >> np.uint32(32 - d))

    rotations = [(13, 15, 26, 6), (17, 29, 16, 24)]
    x0 = (x0 + ks0).astype(np.uint32)
    x1 = (x1 + ks1).astype(np.uint32)
    ks = (ks1, ks2, ks0)
    for i in range(5):
        for r in rotations[i % 2]:
            x0 = (x0 + x1).astype(np.uint32)
            x1 = rotl(x1, r)
            x1 = x0 ^ x1
        x0 = (x0 + ks[i % 3]).astype(np.uint32)
        x1 = (x1 + ks[(i + 1) % 3] + np.uint32(i + 1)).astype(np.uint32)
    return x0, x1


def _np_uniform(seed, n):
    key = (np.uint32(0), np.uint32(seed))
    c1 = np.zeros(n, dtype=np.uint32)
    c2 = np.arange(n, dtype=np.uint32)
    b1, b2 = _np_threefry2x32(key, (c1, c2))
    bits = b1 ^ b2
    floats = ((bits >> np.uint32(9)) | np.uint32(0x3F800000)).view(np.float32)
    return floats - np.float32(1.0)


_PERM = np.argsort(_np_uniform(42, _B), kind="stable")
_SRC_ROWS = np.concatenate([_PERM, np.arange(_B, _NR)]).astype(np.int32)
_RIDX = np.zeros((_NW, 16), np.int32)
_RIDX[:, :_RPW] = _SRC_ROWS.reshape(_NW, _RPW)

_mesh = plsc.VectorSubcoreMesh(core_axis_name="c", subcore_axis_name="s")


_TCH = _NCH


@functools.partial(
    pl.kernel,
    out_type=(
        jax.ShapeDtypeStruct((_NR * _T,), jnp.float32),
    ),
    mesh=_mesh,
    scratch_types=[
        pltpu.VMEM((16,), jnp.int32),
        *([pltpu.VMEM((_CS,), jnp.float32)] * _D),
        *([pltpu.SemaphoreType.DMA] * _D),
        *([pltpu.SemaphoreType.DMA] * _D),
    ],
)
def _remix_copy(src, ridx, out, idx_v, *scratch):
    bufs = scratch[:_D]
    gsems = scratch[_D : 2 * _D]
    ssems = scratch[2 * _D :]
    wid = lax.axis_index("s") * 2 + lax.axis_index("c")
    pltpu.sync_copy(ridx.at[wid], idx_v)
    v = idx_v[...]
    src_off = [v[j] * _T for j in range(_RPW)]
    dst_base = wid * _RPW * _T

    def gather(k):
        j, c = divmod(k, _CPR)
        ref = src.at[pl.ds(src_off[j] + c * _CS, _CS)]
        return pltpu.async_copy(ref, bufs[k % _D], gsems[k % _D])

    def scatter(k):
        ref = out.at[pl.ds(dst_base + k * _CS, _CS)]
        return pltpu.async_copy(bufs[k % _D], ref, ssems[k % _D])

    gd = [None] * _TCH
    sd = [None] * _TCH
    for k in range(_TCH):
        if k >= _D:
            sd[k - _D].wait()
        gd[k] = gather(k)
        if k >= _LAG:
            gd[k - _LAG].wait()
            sd[k - _LAG] = scatter(k - _LAG)
    for k in range(_TCH - _LAG, _TCH):
        gd[k].wait()
        sd[k] = scatter(k)
    for k in range(_TCH - _D, _TCH):
        sd[k].wait()


def kernel(sources, target):
    src = sources.reshape(_NR * _T)
    (out,) = _remix_copy(src, jnp.asarray(_RIDX))
    tout = target * jnp.float32(1.0)
    return out.reshape(2, _B, 1, _T), tout


# speedup vs baseline: 1.0194x; 1.0194x over previous
import functools

import jax
import jax.numpy as jnp
import numpy as np
from jax import lax
from jax.experimental import pallas as pl
from jax.experimental.pallas import tpu as pltpu
from jax.experimental.pallas import tpu_sc as plsc

_B = 32
_T = 160000
_NR = 2 * _B
_NW = 32
_RPW = _NR // _NW
_CS = 16000
_CPR = _T // _CS
_NCH = _RPW * _CPR
_TCH = _NCH + _CPR
_D = 8
_LAG = 4


def _np_threefry2x32(keypair, count):
    x0 = np.uint32(count[0]).copy()
    x1 = np.uint32(count[1]).copy()
    ks0, ks1 = np.uint32(keypair[0]), np.uint32(keypair[1])
    ks2 = np.uint32(0x1BD11BDA) ^ ks0 ^ ks1

    def rotl(x, d):
        return (x << np.uint32(d)) | (x >> np.uint32(32 - d))

    rotations = [(13, 15, 26, 6), (17, 29, 16, 24)]
    x0 = (x0 + ks0).astype(np.uint32)
    x1 = (x1 + ks1).astype(np.uint32)
    ks = (ks1, ks2, ks0)
    for i in range(5):
        for r in rotations[i % 2]:
            x0 = (x0 + x1).astype(np.uint32)
            x1 = rotl(x1, r)
            x1 = x0 ^ x1
        x0 = (x0 + ks[i % 3]).astype(np.uint32)
        x1 = (x1 + ks[(i + 1) % 3] + np.uint32(i + 1)).astype(np.uint32)
    return x0, x1


def _np_uniform(seed, n):
    key = (np.uint32(0), np.uint32(seed))
    c1 = np.zeros(n, dtype=np.uint32)
    c2 = np.arange(n, dtype=np.uint32)
    b1, b2 = _np_threefry2x32(key, (c1, c2))
    bits = b1 ^ b2
    floats = ((bits >> np.uint32(9)) | np.uint32(0x3F800000)).view(np.float32)
    return floats - np.float32(1.0)


_PERM = np.argsort(_np_uniform(42, _B), kind="stable")
_SRC_ROWS = np.concatenate([_PERM, np.arange(_B, _NR)]).astype(np.int32)
_RIDX = np.zeros((_NW, 16), np.int32)
_RIDX[:, :_RPW] = _SRC_ROWS.reshape(_NW, _RPW)

_mesh = plsc.VectorSubcoreMesh(core_axis_name="c", subcore_axis_name="s")


@functools.partial(
    pl.kernel,
    out_type=(
        jax.ShapeDtypeStruct((_NR * _T,), jnp.float32),
        jax.ShapeDtypeStruct((_B * _T,), jnp.float32),
    ),
    mesh=_mesh,
    scratch_types=[
        pltpu.VMEM((16,), jnp.int32),
        *([pltpu.VMEM((_CS,), jnp.float32)] * _D),
        *([pltpu.SemaphoreType.DMA] * _D),
        *([pltpu.SemaphoreType.DMA] * _D),
    ],
)
def _remix_copy(src, tgt, ridx, out, tout, idx_v, *scratch):
    bufs = scratch[:_D]
    gsems = scratch[_D : 2 * _D]
    ssems = scratch[2 * _D :]
    wid = lax.axis_index("s") * 2 + lax.axis_index("c")
    pltpu.sync_copy(ridx.at[wid], idx_v)
    v = idx_v[...]
    src_off = [v[j] * _T for j in range(_RPW)]
    dst_base = wid * _RPW * _T
    tgt_base = wid * _T

    def gather(k):
        if k < _NCH:
            j, c = divmod(k, _CPR)
            ref = src.at[pl.ds(src_off[j] + c * _CS, _CS)]
        else:
            c = k - _NCH
            ref = tgt.at[pl.ds(tgt_base + c * _CS, _CS)]
        return pltpu.async_copy(ref, bufs[k % _D], gsems[k % _D])

    def scatter(k):
        if k < _NCH:
            ref = out.at[pl.ds(dst_base + k * _CS, _CS)]
        else:
            c = k - _NCH
            ref = tout.at[pl.ds(tgt_base + c * _CS, _CS)]
        return pltpu.async_copy(bufs[k % _D], ref, ssems[k % _D])

    gd = [None] * _TCH
    sd = [None] * _TCH
    for k in range(_TCH):
        if k >= _D:
            sd[k - _D].wait()
        gd[k] = gather(k)
        if k >= _LAG:
            gd[k - _LAG].wait()
            sd[k - _LAG] = scatter(k - _LAG)
    for k in range(_TCH - _LAG, _TCH):
        gd[k].wait()
        sd[k] = scatter(k)
    for k in range(_TCH - _D, _TCH):
        sd[k].wait()


def kernel(sources, target):
    src = sources.reshape(_NR * _T)
    tgt = target.reshape(_B * _T)
    out, tout = _remix_copy(src, tgt, jnp.asarray(_RIDX))
    return out.reshape(2, _B, 1, _T), tout.reshape(_B, 1, _T)
